# baseline (speedup 1.0000x reference)
import functools
import math

import jax
import jax.numpy as jnp
from jax import lax
from jax.experimental import pallas as pl
from jax.experimental.pallas import tpu as pltpu
from jax.experimental.pallas import tpu_sc as plsc

N = 10000
E = 320000
D = 128
H = 4
DH = 32
PD = 16
R = 16
L = 2
ALPHA = 0.5

NB = 10
BN = N // NB
EB = 160
BE = E // EB
DR = 320

NC = 2
NS = 16
NW = NC * NS
GB = 80

_INV_SQRT_DH = 1.0 / math.sqrt(DH)


def _qkv_body(hs, hm, wqs, wks, wvs, wqm, wkm, wvm, wes, wem, rel,
              tdst, tsrc, rp):
    a = hs[...]
    b = hm[...]
    dot = functools.partial(jnp.dot, preferred_element_type=jnp.float32)
    tdst[...] = jnp.concatenate(
        [dot(a, wqs[...]), dot(b, wqm[...])], axis=1).astype(jnp.bfloat16)
    tsrc[...] = jnp.concatenate(
        [dot(a, wks[...]), dot(a, wvs[...]), dot(b, wkm[...]), dot(b, wvm[...])],
        axis=1).astype(jnp.bfloat16)
    rp[...] = jnp.concatenate([dot(rel[...], wes[...]), dot(rel[...], wem[...])],
                              axis=1)


def _qkv_call(hs, hm, wqs, wks, wvs, wqm, wkm, wvm, wes, wem, rel):
    w_spec = pl.BlockSpec((D, D), lambda i: (0, 0))
    we_spec = pl.BlockSpec((PD, D), lambda i: (0, 0))
    return pl.pallas_call(
        _qkv_body,
        grid=(NB,),
        in_specs=[
            pl.BlockSpec((BN, D), lambda i: (i, 0)),
            pl.BlockSpec((BN, D), lambda i: (i, 0)),
            w_spec, w_spec, w_spec, w_spec, w_spec, w_spec,
            we_spec, we_spec,
            pl.BlockSpec((R, PD), lambda i: (0, 0)),
        ],
        out_specs=[
            pl.BlockSpec((BN, 2 * D), lambda i: (i, 0)),
            pl.BlockSpec((BN, 4 * D), lambda i: (i, 0)),
            pl.BlockSpec((R, 2 * D), lambda i: (0, 0)),
        ],
        out_shape=[
            jax.ShapeDtypeStruct((N, 2 * D), jnp.bfloat16),
            jax.ShapeDtypeStruct((N, 4 * D), jnp.bfloat16),
            jax.ShapeDtypeStruct((R, 2 * D), jnp.float32),
        ],
    )(hs, hm, wqs, wks, wvs, wqm, wkm, wvm, wes, wem, rel)


def _gather_call(dst1, src1, tdst, tsrc):
    epw = E // NW
    nch = epw // GB
    mesh = plsc.VectorSubcoreMesh(core_axis_name="c", subcore_axis_name="s")

    @functools.partial(
        pl.kernel,
        mesh=mesh,
        out_type=[
            jax.ShapeDtypeStruct((E, D), jnp.float32),
            jax.ShapeDtypeStruct((E, 2 * D), jnp.float32),
        ],
        scratch_types=[
            pltpu.VMEM((epw,), jnp.int32),
            pltpu.VMEM((epw,), jnp.int32),
            pltpu.VMEM((GB, D), jnp.float32),
            pltpu.VMEM((GB, 2 * D), jnp.float32),
            pltpu.SemaphoreType.DMA,
        ])
    def gk(dst_h, src_h, tdst_h, tsrc_h, qd_h, kv_h, dv, sv, qb, kb, sem):
        wid = lax.axis_index("s") * NC + lax.axis_index("c")
        base = wid * epw
        pltpu.sync_copy(dst_h.at[pl.ds(base, epw)], dv)
        pltpu.sync_copy(src_h.at[pl.ds(base, epw)], sv)

        def body(ci, carry):
            off = pl.multiple_of(ci * GB, GB)
            c1 = pltpu.async_copy(tdst_h.at[dv.at[pl.ds(off, GB)]], qb, sem)
            c2 = pltpu.async_copy(tsrc_h.at[sv.at[pl.ds(off, GB)]], kb, sem)
            c1.wait()
            c2.wait()
            pltpu.sync_copy(qb, qd_h.at[pl.ds(base + off, GB)])
            pltpu.sync_copy(kb, kv_h.at[pl.ds(base + off, GB)])
            return carry

        lax.fori_loop(0, nch, body, 0)

    return gk(dst1, src1, tdst, tsrc)


NHI = 80


def _edge_body(qd, kv, rp, et, dt, vs_out, vm_out, as_out, am_out):
    t = et[0, 0, :]
    oh = (t[:, None] == lax.broadcasted_iota(jnp.int32, (BE, R), 1)
          ).astype(jnp.float32)
    e2 = jnp.dot(oh, rp[...], preferred_element_type=jnp.float32)
    qd_all = qd[...].astype(jnp.float32)
    kv_all = kv[...].astype(jnp.float32)
    d = dt[0, 0, :]
    lo = d & (D - 1)
    hi = lax.shift_right_logical(d, 7)
    a = (lo[:, None] == lax.broadcasted_iota(jnp.int32, (BE, D), 1)
         ).astype(jnp.float32)
    b = (hi[:, None] == lax.broadcasted_iota(jnp.int32, (BE, NHI), 1)
         ).astype(jnp.float32)

    @pl.when(pl.program_id(0) == 0)
    def _():
        as_out[...] = jnp.zeros((D, H * NHI), jnp.float32)
        am_out[...] = jnp.zeros((D, H * NHI), jnp.float32)

    def stream(qoff, koff, voff, eoff, out_ref, acc_ref):
        q = qd_all[:, qoff:qoff + D]
        e = e2[:, eoff:eoff + D]
        k = kv_all[:, koff:koff + D] + e
        v = kv_all[:, voff:voff + D] + e
        prod = q * k
        wcols = []
        dcols = []
        for h in range(H):
            sl = slice(h * DH, (h + 1) * DH)
            sh = jnp.sum(prod[:, sl], axis=1, keepdims=True) * _INV_SQRT_DH
            eh = jnp.exp(sh)
            wcols.append(eh * v[:, sl])
            dcols.append(lax.dot_general(
                a * eh, b, dimension_numbers=(((0,), (0,)), ((), ())),
                preferred_element_type=jnp.float32))
        out_ref[...] = jnp.concatenate(wcols, axis=1)
        acc_ref[...] += jnp.concatenate(dcols, axis=1)

    stream(0, 0, D, 0, vs_out, as_out)
    stream(D, 2 * D, 3 * D, D, vm_out, am_out)


def _edge_call(qd, kv, rp, et3, dt3):
    return pl.pallas_call(
        _edge_body,
        grid=(EB,),
        in_specs=[
            pl.BlockSpec((BE, 2 * D), lambda i: (i, 0)),
            pl.BlockSpec((BE, 4 * D), lambda i: (i, 0)),
            pl.BlockSpec((R, 2 * D), lambda i: (0, 0)),
            pl.BlockSpec((1, 1, BE), lambda i: (i, 0, 0)),
            pl.BlockSpec((1, 1, BE), lambda i: (i, 0, 0)),
        ],
        out_specs=[
            pl.BlockSpec((BE, D), lambda i: (i, 0)),
            pl.BlockSpec((BE, D), lambda i: (i, 0)),
            pl.BlockSpec((D, H * NHI), lambda i: (0, 0)),
            pl.BlockSpec((D, H * NHI), lambda i: (0, 0)),
        ],
        out_shape=[
            jax.ShapeDtypeStruct((E, D), jnp.float32),
            jax.ShapeDtypeStruct((E, D), jnp.float32),
            jax.ShapeDtypeStruct((D, H * NHI), jnp.float32),
            jax.ShapeDtypeStruct((D, H * NHI), jnp.float32),
        ],
    )(qd, kv, rp, et3, dt3)


_SC_NCH = 256
_SC_CHUNKS = E // GB
_SC_PAD = NS * _SC_NCH


def _scatter_call(dst2, vals_s, vals_m, zeros_n):
    mesh = plsc.VectorSubcoreMesh(core_axis_name="c", subcore_axis_name="s")

    @functools.partial(
        pl.kernel,
        mesh=mesh,
        out_type=jax.ShapeDtypeStruct((2 * N, D), jnp.float32),
        scratch_types=[
            pltpu.VMEM((_SC_NCH, GB), jnp.int32),
            pltpu.VMEM((GB, D), jnp.float32),
            pltpu.VMEM_SHARED((N, D), jnp.float32),
        ])
    def sk(dst_h, vs_h, vm_h, z_h, outp_h, dv, vb, pay):
        c = lax.axis_index("c")
        s = lax.axis_index("s")

        @pl.when(s == 0)
        def _():
            pltpu.sync_copy(z_h, pay)

        plsc.subcore_barrier()
        row0 = s * _SC_NCH
        pltpu.sync_copy(dst_h.at[pl.ds(row0, _SC_NCH)], dv)
        nch_here = jnp.minimum(_SC_NCH, jnp.maximum(_SC_CHUNKS - row0, 0))

        def make_body(v_h):
            def body(ci, carry):
                off = pl.multiple_of((row0 + ci) * GB, GB)
                pltpu.sync_copy(v_h.at[pl.ds(off, GB)], vb)
                pltpu.sync_copy(vb, pay.at[dv.at[ci]], add=True)
                return carry
            return body

        @pl.when(c == 0)
        def _():
            lax.fori_loop(0, nch_here, make_body(vs_h), 0)

        @pl.when(c == 1)
        def _():
            lax.fori_loop(0, nch_here, make_body(vm_h), 0)

        plsc.subcore_barrier()

        @pl.when(s == 0)
        def _():
            pltpu.sync_copy(pay, outp_h.at[pl.ds(c * N, N)])

    return sk(dst2, vals_s, vals_m, zeros_n)


def _combine_body(os_ref, om_ref, ds_ref, dm_ref, hs_ref, hm_ref, wos, wom,
                  hs_out, hm_out):
    def stream(o_ref, d_ref, h_ref, wo, out_ref):
        o = o_ref[...]
        d = d_ref[...]
        cols = []
        for h in range(H):
            dh = d[:, h:h + 1]
            cols.append(o[:, h * DH:(h + 1) * DH] / (dh + 1e-9))
        agg = jnp.concatenate(cols, axis=1)
        out_ref[...] = (jnp.dot(agg, wo[...], preferred_element_type=jnp.float32)
                        + h_ref[...])

    stream(os_ref, ds_ref, hs_ref, wos, hs_out)
    stream(om_ref, dm_ref, hm_ref, wom, hm_out)


def _combine_call(outs_s, outs_m, den_s, den_m, hs, hm, wos, wom):
    blk = pl.BlockSpec((BN, D), lambda i: (i, 0))
    dblk = pl.BlockSpec((BN, H), lambda i: (i, 0))
    wblk = pl.BlockSpec((D, D), lambda i: (0, 0))
    return pl.pallas_call(
        _combine_body,
        grid=(NB,),
        in_specs=[blk, blk, dblk, dblk, blk, blk, wblk, wblk],
        out_specs=[blk, blk],
        out_shape=[
            jax.ShapeDtypeStruct((N, D), jnp.float32),
            jax.ShapeDtypeStruct((N, D), jnp.float32),
        ],
    )(outs_s, outs_m, den_s, den_m, hs, hm, wos, wom)


def _head_body(hs_ref, hm_ref, cent_ref, wout, params, out_ref):
    b = params[0]
    gamma = params[1]
    beta = params[2]
    ls = jnp.dot(hs_ref[...], wout[...], preferred_element_type=jnp.float32) + b
    lm = jnp.dot(hm_ref[...], wout[...], preferred_element_type=jnp.float32) + b
    lg = ALPHA * ls + (1.0 - ALPHA) * lm
    scale = cent_ref[...] * gamma + beta
    out_ref[...] = jnp.maximum(scale * lg, 0.0)


def _head_call(hs, hm, cent2, wout, params):
    hblk = pl.BlockSpec((BN, D), lambda i: (i, 0))
    return pl.pallas_call(
        _head_body,
        grid=(NB,),
        in_specs=[
            hblk, hblk,
            pl.BlockSpec((BN, 1), lambda i: (i, 0)),
            pl.BlockSpec((D, 1), lambda i: (0, 0)),
            pl.BlockSpec(memory_space=pltpu.SMEM),
        ],
        out_specs=pl.BlockSpec((BN, 1), lambda i: (i, 0)),
        out_shape=jax.ShapeDtypeStruct((N, 1), jnp.float32),
    )(hs, hm, cent2, wout, params)


def kernel(feats_struct, feats_semantic, edge_types, edge_index, centrality,
           rel_emb, Wq_s, Wk_s, Wv_s, We_s, Wo_s, Wq_m, Wk_m, Wv_m, We_m,
           Wo_m, W_out, b_out, gamma, beta):
    dst1 = edge_index[1]
    src1 = edge_index[0]
    dst2 = jnp.pad(dst1, (0, _SC_PAD * GB - E)).reshape(_SC_PAD, GB)
    et3 = edge_types.reshape(EB, 1, BE)
    dt3 = dst1.reshape(EB, 1, BE)
    zeros_n = jnp.zeros((N, D), jnp.float32)
    params = jnp.concatenate([b_out, gamma, beta]).astype(jnp.float32)
    cent2 = centrality.reshape(N, 1)

    def unpack_den(acc):
        a = acc.reshape(D, H, NHI)
        return a.transpose(2, 0, 1).reshape(NHI * D, H)[:N]

    hs, hm = feats_struct, feats_semantic
    for l in range(L):
        tdst, tsrc, rp = _qkv_call(hs, hm, Wq_s[l], Wk_s[l], Wv_s[l],
                                   Wq_m[l], Wk_m[l], Wv_m[l],
                                   We_s[l], We_m[l], rel_emb)
        tdst32 = lax.bitcast_convert_type(
            tdst.reshape(N, D, 2), jnp.float32)
        tsrc32 = lax.bitcast_convert_type(
            tsrc.reshape(N, 2 * D, 2), jnp.float32)
        qd32, kv32 = _gather_call(dst1, src1, tdst32, tsrc32)
        qd = lax.bitcast_convert_type(qd32, jnp.bfloat16).reshape(E, 2 * D)
        kv = lax.bitcast_convert_type(kv32, jnp.bfloat16).reshape(E, 4 * D)
        vals_s, vals_m, acc_s, acc_m = _edge_call(qd, kv, rp, et3, dt3)
        outp = _scatter_call(dst2, vals_s, vals_m, zeros_n)
        hs, hm = _combine_call(outp[:N], outp[N:],
                               unpack_den(acc_s), unpack_den(acc_m),
                               hs, hm, Wo_s[l], Wo_m[l])

    return _head_call(hs, hm, cent2, W_out, params)


# speedup vs baseline: 2.9791x; 2.9791x over previous
import functools
import math

import jax
import jax.numpy as jnp
from jax import lax
from jax.experimental import pallas as pl
from jax.experimental.pallas import tpu as pltpu
from jax.experimental.pallas import tpu_sc as plsc

N = 10000
E = 320000
D = 128
H = 4
DH = 32
PD = 16
R = 16
L = 2
ALPHA = 0.5

NB = 10
BN = N // NB
EB = 160
BE = E // EB
DR = 320

NC = 2
NS = 16
NW = NC * NS
GB = 80

_INV_SQRT_DH = 1.0 / math.sqrt(DH)


def _pack2(a, b):
    ua = lax.bitcast_convert_type(a, jnp.uint32)
    ub = lax.bitcast_convert_type(b, jnp.uint32)
    hi = (ua + jnp.uint32(0x8000)) & jnp.uint32(0xFFFF0000)
    lo = lax.shift_right_logical(ub + jnp.uint32(0x8000), jnp.uint32(16))
    return lax.bitcast_convert_type(hi | lo, jnp.float32)


def _unpack2(w):
    u = lax.bitcast_convert_type(w, jnp.uint32)
    a = lax.bitcast_convert_type(u & jnp.uint32(0xFFFF0000), jnp.float32)
    b = lax.bitcast_convert_type(lax.shift_left(u, jnp.uint32(16)), jnp.float32)
    return a, b


def _qkv_body(hs, hm, wqs, wks, wvs, wqm, wkm, wvm, wes, wem, rel,
              tdst, tsrc, rp):
    a = hs[...]
    b = hm[...]
    dot = functools.partial(jnp.dot, preferred_element_type=jnp.float32)
    tdst[...] = _pack2(dot(a, wqs[...]), dot(b, wqm[...]))
    tsrc[...] = jnp.concatenate(
        [_pack2(dot(a, wks[...]), dot(b, wkm[...])),
         _pack2(dot(a, wvs[...]), dot(b, wvm[...]))], axis=1)
    rp[...] = jnp.concatenate([dot(rel[...], wes[...]), dot(rel[...], wem[...])],
                              axis=1)


def _qkv_call(hs, hm, wqs, wks, wvs, wqm, wkm, wvm, wes, wem, rel):
    w_spec = pl.BlockSpec((D, D), lambda i: (0, 0))
    we_spec = pl.BlockSpec((PD, D), lambda i: (0, 0))
    return pl.pallas_call(
        _qkv_body,
        grid=(NB,),
        in_specs=[
            pl.BlockSpec((BN, D), lambda i: (i, 0)),
            pl.BlockSpec((BN, D), lambda i: (i, 0)),
            w_spec, w_spec, w_spec, w_spec, w_spec, w_spec,
            we_spec, we_spec,
            pl.BlockSpec((R, PD), lambda i: (0, 0)),
        ],
        out_specs=[
            pl.BlockSpec((BN, D), lambda i: (i, 0)),
            pl.BlockSpec((BN, 2 * D), lambda i: (i, 0)),
            pl.BlockSpec((R, 2 * D), lambda i: (0, 0)),
        ],
        out_shape=[
            jax.ShapeDtypeStruct((N, D), jnp.float32),
            jax.ShapeDtypeStruct((N, 2 * D), jnp.float32),
            jax.ShapeDtypeStruct((R, 2 * D), jnp.float32),
        ],
    )(hs, hm, wqs, wks, wvs, wqm, wkm, wvm, wes, wem, rel)


def _gather_call(dst1, src1, tdst, tsrc):
    epw = E // NW
    nch = epw // GB
    mesh = plsc.VectorSubcoreMesh(core_axis_name="c", subcore_axis_name="s")

    @functools.partial(
        pl.kernel,
        mesh=mesh,
        out_type=[
            jax.ShapeDtypeStruct((E, D), jnp.float32),
            jax.ShapeDtypeStruct((E, 2 * D), jnp.float32),
        ],
        scratch_types=[
            pltpu.VMEM((epw,), jnp.int32),
            pltpu.VMEM((epw,), jnp.int32),
            pltpu.VMEM((GB, D), jnp.float32),
            pltpu.VMEM((GB, 2 * D), jnp.float32),
            pltpu.SemaphoreType.DMA,
        ])
    def gk(dst_h, src_h, tdst_h, tsrc_h, qd_h, kv_h, dv, sv, qb, kb, sem):
        wid = lax.axis_index("s") * NC + lax.axis_index("c")
        base = wid * epw
        pltpu.sync_copy(dst_h.at[pl.ds(base, epw)], dv)
        pltpu.sync_copy(src_h.at[pl.ds(base, epw)], sv)

        def body(ci, carry):
            off = pl.multiple_of(ci * GB, GB)
            c1 = pltpu.async_copy(tdst_h.at[dv.at[pl.ds(off, GB)]], qb, sem)
            c2 = pltpu.async_copy(tsrc_h.at[sv.at[pl.ds(off, GB)]], kb, sem)
            c1.wait()
            c2.wait()
            pltpu.sync_copy(qb, qd_h.at[pl.ds(base + off, GB)])
            pltpu.sync_copy(kb, kv_h.at[pl.ds(base + off, GB)])
            return carry

        lax.fori_loop(0, nch, body, 0)

    return gk(dst1, src1, tdst, tsrc)


NHI = 80


def _edge_body(qd, kv, rp, et, dt, vs_out, vm_out, as_out, am_out):
    t = et[0, 0, :]
    oh = (t[:, None] == lax.broadcasted_iota(jnp.int32, (BE, R), 1)
          ).astype(jnp.float32)
    e2 = jnp.dot(oh, rp[...], preferred_element_type=jnp.float32)
    q_s, q_m = _unpack2(qd[...])
    kv_all = kv[...]
    k_s, k_m = _unpack2(kv_all[:, :D])
    v_s, v_m = _unpack2(kv_all[:, D:])
    d = dt[0, 0, :]
    lo = d & (D - 1)
    hi = lax.shift_right_logical(d, 7)
    a = (lo[:, None] == lax.broadcasted_iota(jnp.int32, (BE, D), 1)
         ).astype(jnp.float32)
    b = (hi[:, None] == lax.broadcasted_iota(jnp.int32, (BE, NHI), 1)
         ).astype(jnp.float32)

    @pl.when(pl.program_id(0) == 0)
    def _():
        as_out[...] = jnp.zeros((D, H * NHI), jnp.float32)
        am_out[...] = jnp.zeros((D, H * NHI), jnp.float32)

    def stream(q, k0, v0, eoff, out_ref, acc_ref):
        e = e2[:, eoff:eoff + D]
        k = k0 + e
        v = v0 + e
        prod = q * k
        wcols = []
        dcols = []
        for h in range(H):
            sl = slice(h * DH, (h + 1) * DH)
            sh = jnp.sum(prod[:, sl], axis=1, keepdims=True) * _INV_SQRT_DH
            eh = jnp.exp(sh)
            wcols.append(eh * v[:, sl])
            dcols.append(lax.dot_general(
                a * eh, b, dimension_numbers=(((0,), (0,)), ((), ())),
                preferred_element_type=jnp.float32))
        out_ref[...] = jnp.concatenate(wcols, axis=1)
        acc_ref[...] += jnp.concatenate(dcols, axis=1)

    stream(q_s, k_s, v_s, 0, vs_out, as_out)
    stream(q_m, k_m, v_m, D, vm_out, am_out)


def _edge_call(qd, kv, rp, et3, dt3):
    return pl.pallas_call(
        _edge_body,
        grid=(EB,),
        in_specs=[
            pl.BlockSpec((BE, D), lambda i: (i, 0)),
            pl.BlockSpec((BE, 2 * D), lambda i: (i, 0)),
            pl.BlockSpec((R, 2 * D), lambda i: (0, 0)),
            pl.BlockSpec((1, 1, BE), lambda i: (i, 0, 0)),
            pl.BlockSpec((1, 1, BE), lambda i: (i, 0, 0)),
        ],
        out_specs=[
            pl.BlockSpec((BE, D), lambda i: (i, 0)),
            pl.BlockSpec((BE, D), lambda i: (i, 0)),
            pl.BlockSpec((D, H * NHI), lambda i: (0, 0)),
            pl.BlockSpec((D, H * NHI), lambda i: (0, 0)),
        ],
        out_shape=[
            jax.ShapeDtypeStruct((E, D), jnp.float32),
            jax.ShapeDtypeStruct((E, D), jnp.float32),
            jax.ShapeDtypeStruct((D, H * NHI), jnp.float32),
            jax.ShapeDtypeStruct((D, H * NHI), jnp.float32),
        ],
    )(qd, kv, rp, et3, dt3)


_SC_NCH = 256
_SC_CHUNKS = E // GB
_SC_PAD = NS * _SC_NCH


def _scatter_call(dst2, vals_s, vals_m, zeros_n):
    mesh = plsc.VectorSubcoreMesh(core_axis_name="c", subcore_axis_name="s")

    @functools.partial(
        pl.kernel,
        mesh=mesh,
        out_type=jax.ShapeDtypeStruct((2 * N, D), jnp.float32),
        scratch_types=[
            pltpu.VMEM((_SC_NCH, GB), jnp.int32),
            pltpu.VMEM((GB, D), jnp.float32),
            pltpu.VMEM_SHARED((N, D), jnp.float32),
        ])
    def sk(dst_h, vs_h, vm_h, z_h, outp_h, dv, vb, pay):
        c = lax.axis_index("c")
        s = lax.axis_index("s")

        @pl.when(s == 0)
        def _():
            pltpu.sync_copy(z_h, pay)

        plsc.subcore_barrier()
        row0 = s * _SC_NCH
        pltpu.sync_copy(dst_h.at[pl.ds(row0, _SC_NCH)], dv)
        nch_here = jnp.minimum(_SC_NCH, jnp.maximum(_SC_CHUNKS - row0, 0))

        def make_body(v_h):
            def body(ci, carry):
                off = pl.multiple_of((row0 + ci) * GB, GB)
                pltpu.sync_copy(v_h.at[pl.ds(off, GB)], vb)
                pltpu.sync_copy(vb, pay.at[dv.at[ci]], add=True)
                return carry
            return body

        @pl.when(c == 0)
        def _():
            lax.fori_loop(0, nch_here, make_body(vs_h), 0)

        @pl.when(c == 1)
        def _():
            lax.fori_loop(0, nch_here, make_body(vm_h), 0)

        plsc.subcore_barrier()

        @pl.when(s == 0)
        def _():
            pltpu.sync_copy(pay, outp_h.at[pl.ds(c * N, N)])

    return sk(dst2, vals_s, vals_m, zeros_n)


def _combine_body(os_ref, om_ref, ds_ref, dm_ref, hs_ref, hm_ref, wos, wom,
                  hs_out, hm_out):
    def stream(o_ref, d_ref, h_ref, wo, out_ref):
        o = o_ref[...]
        d = d_ref[...]
        cols = []
        for h in range(H):
            dh = d[:, h:h + 1]
            cols.append(o[:, h * DH:(h + 1) * DH] / (dh + 1e-9))
        agg = jnp.concatenate(cols, axis=1)
        out_ref[...] = (jnp.dot(agg, wo[...], preferred_element_type=jnp.float32)
                        + h_ref[...])

    stream(os_ref, ds_ref, hs_ref, wos, hs_out)
    stream(om_ref, dm_ref, hm_ref, wom, hm_out)


def _combine_call(outs_s, outs_m, den_s, den_m, hs, hm, wos, wom):
    blk = pl.BlockSpec((BN, D), lambda i: (i, 0))
    dblk = pl.BlockSpec((BN, H), lambda i: (i, 0))
    wblk = pl.BlockSpec((D, D), lambda i: (0, 0))
    return pl.pallas_call(
        _combine_body,
        grid=(NB,),
        in_specs=[blk, blk, dblk, dblk, blk, blk, wblk, wblk],
        out_specs=[blk, blk],
        out_shape=[
            jax.ShapeDtypeStruct((N, D), jnp.float32),
            jax.ShapeDtypeStruct((N, D), jnp.float32),
        ],
    )(outs_s, outs_m, den_s, den_m, hs, hm, wos, wom)


def _head_body(hs_ref, hm_ref, cent_ref, wout, params, out_ref):
    b = params[0]
    gamma = params[1]
    beta = params[2]
    ls = jnp.dot(hs_ref[...], wout[...], preferred_element_type=jnp.float32) + b
    lm = jnp.dot(hm_ref[...], wout[...], preferred_element_type=jnp.float32) + b
    lg = ALPHA * ls + (1.0 - ALPHA) * lm
    scale = cent_ref[...] * gamma + beta
    out_ref[...] = jnp.maximum(scale * lg, 0.0)


def _head_call(hs, hm, cent2, wout, params):
    hblk = pl.BlockSpec((BN, D), lambda i: (i, 0))
    return pl.pallas_call(
        _head_body,
        grid=(NB,),
        in_specs=[
            hblk, hblk,
            pl.BlockSpec((BN, 1), lambda i: (i, 0)),
            pl.BlockSpec((D, 1), lambda i: (0, 0)),
            pl.BlockSpec(memory_space=pltpu.SMEM),
        ],
        out_specs=pl.BlockSpec((BN, 1), lambda i: (i, 0)),
        out_shape=jax.ShapeDtypeStruct((N, 1), jnp.float32),
    )(hs, hm, cent2, wout, params)


def kernel(feats_struct, feats_semantic, edge_types, edge_index, centrality,
           rel_emb, Wq_s, Wk_s, Wv_s, We_s, Wo_s, Wq_m, Wk_m, Wv_m, We_m,
           Wo_m, W_out, b_out, gamma, beta):
    dst1 = edge_index[1]
    src1 = edge_index[0]
    dst2 = jnp.pad(dst1, (0, _SC_PAD * GB - E)).reshape(_SC_PAD, GB)
    et3 = edge_types.reshape(EB, 1, BE)
    dt3 = dst1.reshape(EB, 1, BE)
    zeros_n = jnp.zeros((N, D), jnp.float32)
    params = jnp.concatenate([b_out, gamma, beta]).astype(jnp.float32)
    cent2 = centrality.reshape(N, 1)

    def unpack_den(acc):
        a = acc.reshape(D, H, NHI)
        return a.transpose(2, 0, 1).reshape(NHI * D, H)[:N]

    hs, hm = feats_struct, feats_semantic
    for l in range(L):
        tdst, tsrc, rp = _qkv_call(hs, hm, Wq_s[l], Wk_s[l], Wv_s[l],
                                   Wq_m[l], Wk_m[l], Wv_m[l],
                                   We_s[l], We_m[l], rel_emb)
        qd, kv = _gather_call(dst1, src1, tdst, tsrc)
        vals_s, vals_m, acc_s, acc_m = _edge_call(qd, kv, rp, et3, dt3)
        outp = _scatter_call(dst2, vals_s, vals_m, zeros_n)
        hs, hm = _combine_call(outp[:N], outp[N:],
                               unpack_den(acc_s), unpack_den(acc_m),
                               hs, hm, Wo_s[l], Wo_m[l])

    return _head_call(hs, hm, cent2, W_out, params)


# speedup vs baseline: 2.9992x; 1.0067x over previous
import functools
import math

import jax
import jax.numpy as jnp
from jax import lax
from jax.experimental import pallas as pl
from jax.experimental.pallas import tpu as pltpu
from jax.experimental.pallas import tpu_sc as plsc

N = 10000
E = 320000
D = 128
H = 4
DH = 32
PD = 16
R = 16
L = 2
ALPHA = 0.5

NB = 10
BN = N // NB
EB = 160
BE = E // EB
DR = 320

NC = 2
NS = 16
NW = NC * NS
GB = 80

_INV_SQRT_DH = 1.0 / math.sqrt(DH)


def _pack2(a, b):
    ua = lax.bitcast_convert_type(a, jnp.uint32)
    ub = lax.bitcast_convert_type(b, jnp.uint32)
    hi = (ua + jnp.uint32(0x8000)) & jnp.uint32(0xFFFF0000)
    lo = lax.shift_right_logical(ub + jnp.uint32(0x8000), jnp.uint32(16))
    return lax.bitcast_convert_type(hi | lo, jnp.float32)


def _unpack2(w):
    u = lax.bitcast_convert_type(w, jnp.uint32)
    a = lax.bitcast_convert_type(u & jnp.uint32(0xFFFF0000), jnp.float32)
    b = lax.bitcast_convert_type(lax.shift_left(u, jnp.uint32(16)), jnp.float32)
    return a, b


def _qkv_body(hs, hm, wqs, wks, wvs, wqm, wkm, wvm, wes, wem, rel,
              tdst, tsrc, rp):
    a = hs[...]
    b = hm[...]
    dot = functools.partial(jnp.dot, preferred_element_type=jnp.float32)
    tdst[...] = _pack2(dot(a, wqs[...]), dot(b, wqm[...]))
    tsrc[...] = jnp.concatenate(
        [_pack2(dot(a, wks[...]), dot(b, wkm[...])),
         _pack2(dot(a, wvs[...]), dot(b, wvm[...]))], axis=1)
    rp[...] = jnp.concatenate([dot(rel[...], wes[...]), dot(rel[...], wem[...])],
                              axis=1)


def _qkv_call(hs, hm, wqs, wks, wvs, wqm, wkm, wvm, wes, wem, rel):
    w_spec = pl.BlockSpec((D, D), lambda i: (0, 0))
    we_spec = pl.BlockSpec((PD, D), lambda i: (0, 0))
    return pl.pallas_call(
        _qkv_body,
        grid=(NB,),
        in_specs=[
            pl.BlockSpec((BN, D), lambda i: (i, 0)),
            pl.BlockSpec((BN, D), lambda i: (i, 0)),
            w_spec, w_spec, w_spec, w_spec, w_spec, w_spec,
            we_spec, we_spec,
            pl.BlockSpec((R, PD), lambda i: (0, 0)),
        ],
        out_specs=[
            pl.BlockSpec((BN, D), lambda i: (i, 0)),
            pl.BlockSpec((BN, 2 * D), lambda i: (i, 0)),
            pl.BlockSpec((R, 2 * D), lambda i: (0, 0)),
        ],
        out_shape=[
            jax.ShapeDtypeStruct((N, D), jnp.float32),
            jax.ShapeDtypeStruct((N, 2 * D), jnp.float32),
            jax.ShapeDtypeStruct((R, 2 * D), jnp.float32),
        ],
    )(hs, hm, wqs, wks, wvs, wqm, wkm, wvm, wes, wem, rel)


def _gather_call(dst1, src1, tdst, tsrc):
    epw = E // NW
    nch = epw // GB
    mesh = plsc.VectorSubcoreMesh(core_axis_name="c", subcore_axis_name="s")

    @functools.partial(
        pl.kernel,
        mesh=mesh,
        out_type=[
            jax.ShapeDtypeStruct((E, D), jnp.float32),
            jax.ShapeDtypeStruct((E, 2 * D), jnp.float32),
        ],
        scratch_types=[
            pltpu.VMEM((epw,), jnp.int32),
            pltpu.VMEM((epw,), jnp.int32),
            pltpu.VMEM((GB, D), jnp.float32),
            pltpu.VMEM((GB, 2 * D), jnp.float32),
            pltpu.SemaphoreType.DMA,
        ])
    def gk(dst_h, src_h, tdst_h, tsrc_h, qd_h, kv_h, dv, sv, qb, kb, sem):
        wid = lax.axis_index("s") * NC + lax.axis_index("c")
        base = wid * epw
        pltpu.sync_copy(dst_h.at[pl.ds(base, epw)], dv)
        pltpu.sync_copy(src_h.at[pl.ds(base, epw)], sv)

        def body(ci, carry):
            off = pl.multiple_of(ci * GB, GB)
            c1 = pltpu.async_copy(tdst_h.at[dv.at[pl.ds(off, GB)]], qb, sem)
            c2 = pltpu.async_copy(tsrc_h.at[sv.at[pl.ds(off, GB)]], kb, sem)
            c1.wait()
            c2.wait()
            pltpu.sync_copy(qb, qd_h.at[pl.ds(base + off, GB)])
            pltpu.sync_copy(kb, kv_h.at[pl.ds(base + off, GB)])
            return carry

        lax.fori_loop(0, nch, body, 0)

    return gk(dst1, src1, tdst, tsrc)


NHI = 80


def _edge_body(qd, kv, rp, et, dt, vs_out, vm_out, as_out, am_out):
    t = et[0, 0, :]
    oh = (t[:, None] == lax.broadcasted_iota(jnp.int32, (BE, R), 1)
          ).astype(jnp.float32)
    e2 = jnp.dot(oh, rp[...], preferred_element_type=jnp.float32)
    q_s, q_m = _unpack2(qd[...])
    kv_all = kv[...]
    k_s, k_m = _unpack2(kv_all[:, :D])
    v_s, v_m = _unpack2(kv_all[:, D:])
    d = dt[0, 0, :]
    lo = d & (D - 1)
    hi = lax.shift_right_logical(d, 7)
    a = (lo[:, None] == lax.broadcasted_iota(jnp.int32, (BE, D), 1)
         ).astype(jnp.float32)
    b = (hi[:, None] == lax.broadcasted_iota(jnp.int32, (BE, NHI), 1)
         ).astype(jnp.float32)

    @pl.when(pl.program_id(0) == 0)
    def _():
        as_out[...] = jnp.zeros((D, H * NHI), jnp.float32)
        am_out[...] = jnp.zeros((D, H * NHI), jnp.float32)

    def stream(q, k0, v0, eoff, out_ref, acc_ref):
        e = e2[:, eoff:eoff + D]
        k = k0 + e
        v = v0 + e
        prod = q * k
        wcols = []
        dcols = []
        for h in range(H):
            sl = slice(h * DH, (h + 1) * DH)
            sh = jnp.sum(prod[:, sl], axis=1, keepdims=True) * _INV_SQRT_DH
            eh = jnp.exp(sh)
            wcols.append(eh * v[:, sl])
            dcols.append(lax.dot_general(
                a * eh, b, dimension_numbers=(((0,), (0,)), ((), ())),
                preferred_element_type=jnp.float32))
        out_ref[...] = jnp.concatenate(wcols, axis=1)
        acc_ref[...] += jnp.concatenate(dcols, axis=1)

    stream(q_s, k_s, v_s, 0, vs_out, as_out)
    stream(q_m, k_m, v_m, D, vm_out, am_out)


def _edge_call(qd, kv, rp, et3, dt3):
    return pl.pallas_call(
        _edge_body,
        grid=(EB,),
        in_specs=[
            pl.BlockSpec((BE, D), lambda i: (i, 0)),
            pl.BlockSpec((BE, 2 * D), lambda i: (i, 0)),
            pl.BlockSpec((R, 2 * D), lambda i: (0, 0)),
            pl.BlockSpec((1, 1, BE), lambda i: (i, 0, 0)),
            pl.BlockSpec((1, 1, BE), lambda i: (i, 0, 0)),
        ],
        out_specs=[
            pl.BlockSpec((BE, D), lambda i: (i, 0)),
            pl.BlockSpec((BE, D), lambda i: (i, 0)),
            pl.BlockSpec((D, H * NHI), lambda i: (0, 0)),
            pl.BlockSpec((D, H * NHI), lambda i: (0, 0)),
        ],
        out_shape=[
            jax.ShapeDtypeStruct((E, D), jnp.float32),
            jax.ShapeDtypeStruct((E, D), jnp.float32),
            jax.ShapeDtypeStruct((D, H * NHI), jnp.float32),
            jax.ShapeDtypeStruct((D, H * NHI), jnp.float32),
        ],
    )(qd, kv, rp, et3, dt3)


_SC_NCH = 256
_SC_CHUNKS = E // GB
_SC_PAD = NS * _SC_NCH


def _scatter_call(dst2, vals_s, vals_m, zeros_n):
    mesh = plsc.VectorSubcoreMesh(core_axis_name="c", subcore_axis_name="s")

    @functools.partial(
        pl.kernel,
        mesh=mesh,
        out_type=jax.ShapeDtypeStruct((2 * N, D), jnp.float32),
        scratch_types=[
            pltpu.VMEM((_SC_NCH, GB), jnp.int32),
            pltpu.VMEM((GB, D), jnp.float32),
            pltpu.VMEM_SHARED((N, D), jnp.float32),
        ])
    def sk(dst_h, vs_h, vm_h, z_h, outp_h, dv, vb, pay):
        c = lax.axis_index("c")
        s = lax.axis_index("s")

        @pl.when(s == 0)
        def _():
            pltpu.sync_copy(z_h, pay)

        plsc.subcore_barrier()
        row0 = s * _SC_NCH
        pltpu.sync_copy(dst_h.at[pl.ds(row0, _SC_NCH)], dv)
        nch_here = jnp.minimum(_SC_NCH, jnp.maximum(_SC_CHUNKS - row0, 0))

        def make_body(v_h):
            def body(ci, carry):
                off = pl.multiple_of((row0 + ci) * GB, GB)
                pltpu.sync_copy(v_h.at[pl.ds(off, GB)], vb)
                pltpu.sync_copy(vb, pay.at[dv.at[ci]], add=True)
                return carry
            return body

        @pl.when(c == 0)
        def _():
            lax.fori_loop(0, nch_here, make_body(vs_h), 0)

        @pl.when(c == 1)
        def _():
            lax.fori_loop(0, nch_here, make_body(vm_h), 0)

        plsc.subcore_barrier()

        @pl.when(s == 0)
        def _():
            pltpu.sync_copy(pay, outp_h.at[pl.ds(c * N, N)])

    return sk(dst2, vals_s, vals_m, zeros_n)


def _combine_stream(o, d, h_prev, wo):
    cols = []
    for h in range(H):
        dh = d[:, h:h + 1]
        cols.append(o[:, h * DH:(h + 1) * DH] / (dh + 1e-9))
    agg = jnp.concatenate(cols, axis=1)
    return jnp.dot(agg, wo, preferred_element_type=jnp.float32) + h_prev


def _layer_body(os_ref, om_ref, ds_ref, dm_ref, hs_ref, hm_ref, wos, wom,
                wqs, wks, wvs, wqm, wkm, wvm, wes, wem, rel,
                hs_out, hm_out, tdst, tsrc, rp):
    a = _combine_stream(os_ref[...], ds_ref[...], hs_ref[...], wos[...])
    b = _combine_stream(om_ref[...], dm_ref[...], hm_ref[...], wom[...])
    hs_out[...] = a
    hm_out[...] = b
    dot = functools.partial(jnp.dot, preferred_element_type=jnp.float32)
    tdst[...] = _pack2(dot(a, wqs[...]), dot(b, wqm[...]))
    tsrc[...] = jnp.concatenate(
        [_pack2(dot(a, wks[...]), dot(b, wkm[...])),
         _pack2(dot(a, wvs[...]), dot(b, wvm[...]))], axis=1)
    rp[...] = jnp.concatenate([dot(rel[...], wes[...]), dot(rel[...], wem[...])],
                              axis=1)


def _layer_call(outp, den_s, den_m, hs, hm, wos, wom,
                wqs, wks, wvs, wqm, wkm, wvm, wes, wem, rel):
    blk = pl.BlockSpec((BN, D), lambda i: (i, 0))
    blk_hi = pl.BlockSpec((BN, D), lambda i: (i + NB, 0))
    dblk = pl.BlockSpec((BN, H), lambda i: (i, 0))
    wblk = pl.BlockSpec((D, D), lambda i: (0, 0))
    we_spec = pl.BlockSpec((PD, D), lambda i: (0, 0))
    return pl.pallas_call(
        _layer_body,
        grid=(NB,),
        in_specs=[blk, blk_hi, dblk, dblk, blk, blk, wblk, wblk,
                  wblk, wblk, wblk, wblk, wblk, wblk,
                  we_spec, we_spec, pl.BlockSpec((R, PD), lambda i: (0, 0))],
        out_specs=[
            blk, blk,
            pl.BlockSpec((BN, D), lambda i: (i, 0)),
            pl.BlockSpec((BN, 2 * D), lambda i: (i, 0)),
            pl.BlockSpec((R, 2 * D), lambda i: (0, 0)),
        ],
        out_shape=[
            jax.ShapeDtypeStruct((N, D), jnp.float32),
            jax.ShapeDtypeStruct((N, D), jnp.float32),
            jax.ShapeDtypeStruct((N, D), jnp.float32),
            jax.ShapeDtypeStruct((N, 2 * D), jnp.float32),
            jax.ShapeDtypeStruct((R, 2 * D), jnp.float32),
        ],
    )(outp, outp, den_s, den_m, hs, hm, wos, wom,
      wqs, wks, wvs, wqm, wkm, wvm, wes, wem, rel)


def _tail_body(os_ref, om_ref, ds_ref, dm_ref, hs_ref, hm_ref, wos, wom,
               cent_ref, wout, params, out_ref):
    a = _combine_stream(os_ref[...], ds_ref[...], hs_ref[...], wos[...])
    b = _combine_stream(om_ref[...], dm_ref[...], hm_ref[...], wom[...])
    bias = params[0]
    gamma = params[1]
    beta = params[2]
    ls = jnp.dot(a, wout[...], preferred_element_type=jnp.float32) + bias
    lm = jnp.dot(b, wout[...], preferred_element_type=jnp.float32) + bias
    lg = ALPHA * ls + (1.0 - ALPHA) * lm
    scale = cent_ref[...] * gamma + beta
    out_ref[...] = jnp.maximum(scale * lg, 0.0)


def _tail_call(outp, den_s, den_m, hs, hm, wos, wom, cent2, wout, params):
    blk = pl.BlockSpec((BN, D), lambda i: (i, 0))
    blk_hi = pl.BlockSpec((BN, D), lambda i: (i + NB, 0))
    dblk = pl.BlockSpec((BN, H), lambda i: (i, 0))
    wblk = pl.BlockSpec((D, D), lambda i: (0, 0))
    return pl.pallas_call(
        _tail_body,
        grid=(NB,),
        in_specs=[
            blk, blk_hi, dblk, dblk, blk, blk, wblk, wblk,
            pl.BlockSpec((BN, 1), lambda i: (i, 0)),
            pl.BlockSpec((D, 1), lambda i: (0, 0)),
            pl.BlockSpec(memory_space=pltpu.SMEM),
        ],
        out_specs=pl.BlockSpec((BN, 1), lambda i: (i, 0)),
        out_shape=jax.ShapeDtypeStruct((N, 1), jnp.float32),
    )(outp, outp, den_s, den_m, hs, hm, wos, wom, cent2, wout, params)


def kernel(feats_struct, feats_semantic, edge_types, edge_index, centrality,
           rel_emb, Wq_s, Wk_s, Wv_s, We_s, Wo_s, Wq_m, Wk_m, Wv_m, We_m,
           Wo_m, W_out, b_out, gamma, beta):
    dst1 = edge_index[1]
    src1 = edge_index[0]
    dst2 = jnp.pad(dst1, (0, _SC_PAD * GB - E)).reshape(_SC_PAD, GB)
    et3 = edge_types.reshape(EB, 1, BE)
    dt3 = dst1.reshape(EB, 1, BE)
    zeros_n = jnp.zeros((N, D), jnp.float32)
    params = jnp.concatenate([b_out, gamma, beta]).astype(jnp.float32)
    cent2 = centrality.reshape(N, 1)

    def unpack_den(acc):
        a = acc.reshape(D, H, NHI)
        return a.transpose(2, 0, 1).reshape(NHI * D, H)[:N]

    hs, hm = feats_struct, feats_semantic
    tdst, tsrc, rp = _qkv_call(hs, hm, Wq_s[0], Wk_s[0], Wv_s[0],
                               Wq_m[0], Wk_m[0], Wv_m[0],
                               We_s[0], We_m[0], rel_emb)
    for l in range(L):
        qd, kv = _gather_call(dst1, src1, tdst, tsrc)
        vals_s, vals_m, acc_s, acc_m = _edge_call(qd, kv, rp, et3, dt3)
        outp = _scatter_call(dst2, vals_s, vals_m, zeros_n)
        den_s = unpack_den(acc_s)
        den_m = unpack_den(acc_m)
        if l + 1 < L:
            hs, hm, tdst, tsrc, rp = _layer_call(
                outp, den_s, den_m, hs, hm, Wo_s[l], Wo_m[l],
                Wq_s[l + 1], Wk_s[l + 1], Wv_s[l + 1],
                Wq_m[l + 1], Wk_m[l + 1], Wv_m[l + 1],
                We_s[l + 1], We_m[l + 1], rel_emb)
        else:
            return _tail_call(outp, den_s, den_m, hs, hm, Wo_s[l], Wo_m[l],
                              cent2, W_out, params)


# speedup vs baseline: 3.1266x; 1.0425x over previous
import functools
import math

import jax
import jax.numpy as jnp
from jax import lax
from jax.experimental import pallas as pl
from jax.experimental.pallas import tpu as pltpu
from jax.experimental.pallas import tpu_sc as plsc

N = 10000
E = 320000
D = 128
H = 4
DH = 32
PD = 16
R = 16
L = 2
ALPHA = 0.5

NB = 10
BN = N // NB
EB = 160
BE = E // EB
DR = 320

NC = 2
NS = 16
NW = NC * NS
GB = 80

_INV_SQRT_DH = 1.0 / math.sqrt(DH)


def _pack2(a, b):
    ua = lax.bitcast_convert_type(a, jnp.uint32)
    ub = lax.bitcast_convert_type(b, jnp.uint32)
    hi = (ua + jnp.uint32(0x8000)) & jnp.uint32(0xFFFF0000)
    lo = lax.shift_right_logical(ub + jnp.uint32(0x8000), jnp.uint32(16))
    return lax.bitcast_convert_type(hi | lo, jnp.float32)


def _unpack2(w):
    u = lax.bitcast_convert_type(w, jnp.uint32)
    a = lax.bitcast_convert_type(u & jnp.uint32(0xFFFF0000), jnp.float32)
    b = lax.bitcast_convert_type(lax.shift_left(u, jnp.uint32(16)), jnp.float32)
    return a, b


def _qkv_body(hs, hm, wqs, wks, wvs, wqm, wkm, wvm, wes, wem, rel,
              tdst, tsrc, rp):
    a = hs[...]
    b = hm[...]
    dot = functools.partial(jnp.dot, preferred_element_type=jnp.float32)
    tdst[...] = _pack2(dot(a, wqs[...]), dot(b, wqm[...]))
    tsrc[...] = jnp.concatenate(
        [_pack2(dot(a, wks[...]), dot(b, wkm[...])),
         _pack2(dot(a, wvs[...]), dot(b, wvm[...]))], axis=1)
    rp[...] = jnp.concatenate([dot(rel[...], wes[...]), dot(rel[...], wem[...])],
                              axis=1)


def _qkv_call(hs, hm, wqs, wks, wvs, wqm, wkm, wvm, wes, wem, rel):
    w_spec = pl.BlockSpec((D, D), lambda i: (0, 0))
    we_spec = pl.BlockSpec((PD, D), lambda i: (0, 0))
    return pl.pallas_call(
        _qkv_body,
        grid=(NB,),
        in_specs=[
            pl.BlockSpec((BN, D), lambda i: (i, 0)),
            pl.BlockSpec((BN, D), lambda i: (i, 0)),
            w_spec, w_spec, w_spec, w_spec, w_spec, w_spec,
            we_spec, we_spec,
            pl.BlockSpec((R, PD), lambda i: (0, 0)),
        ],
        out_specs=[
            pl.BlockSpec((BN, D), lambda i: (i, 0)),
            pl.BlockSpec((BN, 2 * D), lambda i: (i, 0)),
            pl.BlockSpec((R, 2 * D), lambda i: (0, 0)),
        ],
        out_shape=[
            jax.ShapeDtypeStruct((N, D), jnp.float32),
            jax.ShapeDtypeStruct((N, 2 * D), jnp.float32),
            jax.ShapeDtypeStruct((R, 2 * D), jnp.float32),
        ],
    )(hs, hm, wqs, wks, wvs, wqm, wkm, wvm, wes, wem, rel)


def _gather_call(dst1, src1, tdst, tsrc):
    epw = E // NW
    nch = epw // GB
    mesh = plsc.VectorSubcoreMesh(core_axis_name="c", subcore_axis_name="s")

    @functools.partial(
        pl.kernel,
        mesh=mesh,
        out_type=[
            jax.ShapeDtypeStruct((E, D), jnp.float32),
            jax.ShapeDtypeStruct((E, 2 * D), jnp.float32),
        ],
        scratch_types=[
            pltpu.VMEM((epw,), jnp.int32),
            pltpu.VMEM((epw,), jnp.int32),
            pltpu.VMEM((GB, D), jnp.float32),
            pltpu.VMEM((GB, 2 * D), jnp.float32),
            pltpu.SemaphoreType.DMA,
        ])
    def gk(dst_h, src_h, tdst_h, tsrc_h, qd_h, kv_h, dv, sv, qb, kb, sem):
        wid = lax.axis_index("s") * NC + lax.axis_index("c")
        base = wid * epw
        pltpu.sync_copy(dst_h.at[pl.ds(base, epw)], dv)
        pltpu.sync_copy(src_h.at[pl.ds(base, epw)], sv)

        def body(ci, carry):
            off = pl.multiple_of(ci * GB, GB)
            c1 = pltpu.async_copy(tdst_h.at[dv.at[pl.ds(off, GB)]], qb, sem)
            c2 = pltpu.async_copy(tsrc_h.at[sv.at[pl.ds(off, GB)]], kb, sem)
            c1.wait()
            c2.wait()
            pltpu.sync_copy(qb, qd_h.at[pl.ds(base + off, GB)])
            pltpu.sync_copy(kb, kv_h.at[pl.ds(base + off, GB)])
            return carry

        lax.fori_loop(0, nch, body, 0)

    return gk(dst1, src1, tdst, tsrc)


NHI = 80


def _edge_body(qd, kv, rp, et, dt, vs_out, vm_out, as_out, am_out):
    t = et[0, 0, :]
    oh = (t[:, None] == lax.broadcasted_iota(jnp.int32, (BE, R), 1)
          ).astype(jnp.float32)
    e2 = jnp.dot(oh, rp[...], preferred_element_type=jnp.float32)
    q_s, q_m = _unpack2(qd[...])
    kv_all = kv[...]
    k_s, k_m = _unpack2(kv_all[:, :D])
    v_s, v_m = _unpack2(kv_all[:, D:])
    d = dt[0, 0, :]
    lo = d & (D - 1)
    hi = lax.shift_right_logical(d, 7)
    a_t = (lo[None, :] == lax.broadcasted_iota(jnp.int32, (D, BE), 0)
           ).astype(jnp.float32)
    b = (hi[:, None] == lax.broadcasted_iota(jnp.int32, (BE, NHI), 1)
         ).astype(jnp.float32)

    @pl.when(pl.program_id(0) == 0)
    def _():
        as_out[...] = jnp.zeros((D, H * NHI), jnp.float32)
        am_out[...] = jnp.zeros((D, H * NHI), jnp.float32)

    def stream(q, k0, v0, eoff, out_ref, acc_ref):
        e = e2[:, eoff:eoff + D]
        k = k0 + e
        v = v0 + e
        prod = q * k
        wcols = []
        bcols = []
        for h in range(H):
            sl = slice(h * DH, (h + 1) * DH)
            sh = jnp.sum(prod[:, sl], axis=1, keepdims=True) * _INV_SQRT_DH
            eh = jnp.exp(sh)
            wcols.append(eh * v[:, sl])
            bcols.append(eh * b)
        out_ref[...] = jnp.concatenate(wcols, axis=1)
        acc_ref[...] += jnp.dot(a_t, jnp.concatenate(bcols, axis=1),
                                preferred_element_type=jnp.float32)

    stream(q_s, k_s, v_s, 0, vs_out, as_out)
    stream(q_m, k_m, v_m, D, vm_out, am_out)


def _edge_call(qd, kv, rp, et3, dt3):
    return pl.pallas_call(
        _edge_body,
        grid=(EB,),
        in_specs=[
            pl.BlockSpec((BE, D), lambda i: (i, 0)),
            pl.BlockSpec((BE, 2 * D), lambda i: (i, 0)),
            pl.BlockSpec((R, 2 * D), lambda i: (0, 0)),
            pl.BlockSpec((1, 1, BE), lambda i: (i, 0, 0)),
            pl.BlockSpec((1, 1, BE), lambda i: (i, 0, 0)),
        ],
        out_specs=[
            pl.BlockSpec((BE, D), lambda i: (i, 0)),
            pl.BlockSpec((BE, D), lambda i: (i, 0)),
            pl.BlockSpec((D, H * NHI), lambda i: (0, 0)),
            pl.BlockSpec((D, H * NHI), lambda i: (0, 0)),
        ],
        out_shape=[
            jax.ShapeDtypeStruct((E, D), jnp.float32),
            jax.ShapeDtypeStruct((E, D), jnp.float32),
            jax.ShapeDtypeStruct((D, H * NHI), jnp.float32),
            jax.ShapeDtypeStruct((D, H * NHI), jnp.float32),
        ],
    )(qd, kv, rp, et3, dt3)


_SC_NCH = 256
_SC_CHUNKS = E // GB
_SC_PAD = NS * _SC_NCH


def _scatter_call(dst2, vals_s, vals_m, zeros_n):
    mesh = plsc.VectorSubcoreMesh(core_axis_name="c", subcore_axis_name="s")

    @functools.partial(
        pl.kernel,
        mesh=mesh,
        out_type=jax.ShapeDtypeStruct((2 * N, D), jnp.float32),
        scratch_types=[
            pltpu.VMEM((_SC_NCH, GB), jnp.int32),
            pltpu.VMEM((GB, D), jnp.float32),
            pltpu.VMEM_SHARED((N, D), jnp.float32),
        ])
    def sk(dst_h, vs_h, vm_h, z_h, outp_h, dv, vb, pay):
        c = lax.axis_index("c")
        s = lax.axis_index("s")

        @pl.when(s == 0)
        def _():
            pltpu.sync_copy(z_h, pay)

        plsc.subcore_barrier()
        row0 = s * _SC_NCH
        pltpu.sync_copy(dst_h.at[pl.ds(row0, _SC_NCH)], dv)
        nch_here = jnp.minimum(_SC_NCH, jnp.maximum(_SC_CHUNKS - row0, 0))

        def make_body(v_h):
            def body(ci, carry):
                off = pl.multiple_of((row0 + ci) * GB, GB)
                pltpu.sync_copy(v_h.at[pl.ds(off, GB)], vb)
                pltpu.sync_copy(vb, pay.at[dv.at[ci]], add=True)
                return carry
            return body

        @pl.when(c == 0)
        def _():
            lax.fori_loop(0, nch_here, make_body(vs_h), 0)

        @pl.when(c == 1)
        def _():
            lax.fori_loop(0, nch_here, make_body(vm_h), 0)

        plsc.subcore_barrier()

        @pl.when(s == 0)
        def _():
            pltpu.sync_copy(pay, outp_h.at[pl.ds(c * N, N)])

    return sk(dst2, vals_s, vals_m, zeros_n)


def _combine_stream(o, d, h_prev, wo):
    cols = []
    for h in range(H):
        dh = d[:, h:h + 1]
        cols.append(o[:, h * DH:(h + 1) * DH] / (dh + 1e-9))
    agg = jnp.concatenate(cols, axis=1)
    return jnp.dot(agg, wo, preferred_element_type=jnp.float32) + h_prev


def _layer_body(os_ref, om_ref, ds_ref, dm_ref, hs_ref, hm_ref, wos, wom,
                wqs, wks, wvs, wqm, wkm, wvm, wes, wem, rel,
                hs_out, hm_out, tdst, tsrc, rp):
    a = _combine_stream(os_ref[...], ds_ref[...], hs_ref[...], wos[...])
    b = _combine_stream(om_ref[...], dm_ref[...], hm_ref[...], wom[...])
    hs_out[...] = a
    hm_out[...] = b
    dot = functools.partial(jnp.dot, preferred_element_type=jnp.float32)
    tdst[...] = _pack2(dot(a, wqs[...]), dot(b, wqm[...]))
    tsrc[...] = jnp.concatenate(
        [_pack2(dot(a, wks[...]), dot(b, wkm[...])),
         _pack2(dot(a, wvs[...]), dot(b, wvm[...]))], axis=1)
    rp[...] = jnp.concatenate([dot(rel[...], wes[...]), dot(rel[...], wem[...])],
                              axis=1)


def _layer_call(outp, den_s, den_m, hs, hm, wos, wom,
                wqs, wks, wvs, wqm, wkm, wvm, wes, wem, rel):
    blk = pl.BlockSpec((BN, D), lambda i: (i, 0))
    blk_hi = pl.BlockSpec((BN, D), lambda i: (i + NB, 0))
    dblk = pl.BlockSpec((BN, H), lambda i: (i, 0))
    wblk = pl.BlockSpec((D, D), lambda i: (0, 0))
    we_spec = pl.BlockSpec((PD, D), lambda i: (0, 0))
    return pl.pallas_call(
        _layer_body,
        grid=(NB,),
        in_specs=[blk, blk_hi, dblk, dblk, blk, blk, wblk, wblk,
                  wblk, wblk, wblk, wblk, wblk, wblk,
                  we_spec, we_spec, pl.BlockSpec((R, PD), lambda i: (0, 0))],
        out_specs=[
            blk, blk,
            pl.BlockSpec((BN, D), lambda i: (i, 0)),
            pl.BlockSpec((BN, 2 * D), lambda i: (i, 0)),
            pl.BlockSpec((R, 2 * D), lambda i: (0, 0)),
        ],
        out_shape=[
            jax.ShapeDtypeStruct((N, D), jnp.float32),
            jax.ShapeDtypeStruct((N, D), jnp.float32),
            jax.ShapeDtypeStruct((N, D), jnp.float32),
            jax.ShapeDtypeStruct((N, 2 * D), jnp.float32),
            jax.ShapeDtypeStruct((R, 2 * D), jnp.float32),
        ],
    )(outp, outp, den_s, den_m, hs, hm, wos, wom,
      wqs, wks, wvs, wqm, wkm, wvm, wes, wem, rel)


def _tail_body(os_ref, om_ref, ds_ref, dm_ref, hs_ref, hm_ref, wos, wom,
               cent_ref, wout, params, out_ref):
    a = _combine_stream(os_ref[...], ds_ref[...], hs_ref[...], wos[...])
    b = _combine_stream(om_ref[...], dm_ref[...], hm_ref[...], wom[...])
    bias = params[0]
    gamma = params[1]
    beta = params[2]
    ls = jnp.dot(a, wout[...], preferred_element_type=jnp.float32) + bias
    lm = jnp.dot(b, wout[...], preferred_element_type=jnp.float32) + bias
    lg = ALPHA * ls + (1.0 - ALPHA) * lm
    scale = cent_ref[...] * gamma + beta
    out_ref[...] = jnp.maximum(scale * lg, 0.0)


def _tail_call(outp, den_s, den_m, hs, hm, wos, wom, cent2, wout, params):
    blk = pl.BlockSpec((BN, D), lambda i: (i, 0))
    blk_hi = pl.BlockSpec((BN, D), lambda i: (i + NB, 0))
    dblk = pl.BlockSpec((BN, H), lambda i: (i, 0))
    wblk = pl.BlockSpec((D, D), lambda i: (0, 0))
    return pl.pallas_call(
        _tail_body,
        grid=(NB,),
        in_specs=[
            blk, blk_hi, dblk, dblk, blk, blk, wblk, wblk,
            pl.BlockSpec((BN, 1), lambda i: (i, 0)),
            pl.BlockSpec((D, 1), lambda i: (0, 0)),
            pl.BlockSpec(memory_space=pltpu.SMEM),
        ],
        out_specs=pl.BlockSpec((BN, 1), lambda i: (i, 0)),
        out_shape=jax.ShapeDtypeStruct((N, 1), jnp.float32),
    )(outp, outp, den_s, den_m, hs, hm, wos, wom, cent2, wout, params)


def kernel(feats_struct, feats_semantic, edge_types, edge_index, centrality,
           rel_emb, Wq_s, Wk_s, Wv_s, We_s, Wo_s, Wq_m, Wk_m, Wv_m, We_m,
           Wo_m, W_out, b_out, gamma, beta):
    dst1 = edge_index[1]
    src1 = edge_index[0]
    dst2 = jnp.pad(dst1, (0, _SC_PAD * GB - E)).reshape(_SC_PAD, GB)
    et3 = edge_types.reshape(EB, 1, BE)
    dt3 = dst1.reshape(EB, 1, BE)
    zeros_n = jnp.zeros((N, D), jnp.float32)
    params = jnp.concatenate([b_out, gamma, beta]).astype(jnp.float32)
    cent2 = centrality.reshape(N, 1)

    def unpack_den(acc):
        a = acc.reshape(D, H, NHI)
        return a.transpose(2, 0, 1).reshape(NHI * D, H)[:N]

    hs, hm = feats_struct, feats_semantic
    tdst, tsrc, rp = _qkv_call(hs, hm, Wq_s[0], Wk_s[0], Wv_s[0],
                               Wq_m[0], Wk_m[0], Wv_m[0],
                               We_s[0], We_m[0], rel_emb)
    for l in range(L):
        qd, kv = _gather_call(dst1, src1, tdst, tsrc)
        vals_s, vals_m, acc_s, acc_m = _edge_call(qd, kv, rp, et3, dt3)
        outp = _scatter_call(dst2, vals_s, vals_m, zeros_n)
        den_s = unpack_den(acc_s)
        den_m = unpack_den(acc_m)
        if l + 1 < L:
            hs, hm, tdst, tsrc, rp = _layer_call(
                outp, den_s, den_m, hs, hm, Wo_s[l], Wo_m[l],
                Wq_s[l + 1], Wk_s[l + 1], Wv_s[l + 1],
                Wq_m[l + 1], Wk_m[l + 1], Wv_m[l + 1],
                We_s[l + 1], We_m[l + 1], rel_emb)
        else:
            return _tail_call(outp, den_s, den_m, hs, hm, Wo_s[l], Wo_m[l],
                              cent2, W_out, params)
